# baseline (speedup 1.0000x reference)
import functools

import jax
import jax.numpy as jnp
from jax.experimental import pallas as pl
from jax.experimental.pallas import tpu as pltpu

U = 4096
V = 4096
D = 128
H = 128
BLK = 256
NB = U // BLK

_PREC = jax.lax.Precision.HIGHEST


def _dot(a, b):
    return jax.lax.dot_general(
        a, b, (((1,), (0,)), ((), ())),
        precision=_PREC, preferred_element_type=jnp.float32)


def _dgcn_kernel(
    uv_adj_ref, vu_adj_ref, ufea_ref, vfea_ref,
    Wu1_ref, Wv1_ref, Wv2_ref, Wu2_ref,
    ufc1a_ref, ufc1b_ref, ufc1bias_ref,
    vfc1a_ref, vfc1b_ref, vfc1bias_ref,
    ufc2_ref, ufc2bias_ref, vfc2_ref, vfc2bias_ref,
    a_ref,
    hu_ref, hv_ref,
    su_s, sv_s, vu_s, uv_s, tv_s, tu_s,
):
    p = pl.program_id(0)
    b = pl.program_id(1)
    rows = pl.ds(b * BLK, BLK)

    @pl.when(jnp.logical_and(p == 0, b == 0))
    def _init_supports():
        su_s[...] = _dot(ufea_ref[...], Wu1_ref[...])
        sv_s[...] = _dot(vfea_ref[...], Wv1_ref[...])

    @pl.when(p == 0)
    def _phase0():
        vu_s[rows, :] = jnp.maximum(_dot(vu_adj_ref[...], su_s[...]), 0.0)
        uv_s[rows, :] = jnp.maximum(_dot(uv_adj_ref[...], sv_s[...]), 0.0)

    @pl.when(jnp.logical_and(p == 1, b == 0))
    def _init_t():
        tv_s[...] = _dot(vu_s[...], Wv2_ref[...])
        tu_s[...] = _dot(uv_s[...], Wu2_ref[...])

    @pl.when(p == 1)
    def _phase1():
        a = a_ref[0, 0]
        uv2 = jnp.maximum(_dot(uv_adj_ref[...], tv_s[...]), 0.0)
        vu2 = jnp.maximum(_dot(vu_adj_ref[...], tu_s[...]), 0.0)

        hu = _dot(uv2, ufc1a_ref[...]) + _dot(ufea_ref[rows, :], ufc1b_ref[...])
        hu = jnp.maximum(hu + ufc1bias_ref[...], 0.0)
        hu = _dot(hu, ufc2_ref[...]) + ufc2bias_ref[...]
        hu_ref[...] = jnp.where(hu >= 0.0, hu, a * hu)

        hv = _dot(vu2, vfc1a_ref[...]) + _dot(vfea_ref[rows, :], vfc1b_ref[...])
        hv = jnp.maximum(hv + vfc1bias_ref[...], 0.0)
        hv = _dot(hv, vfc2_ref[...]) + vfc2bias_ref[...]
        hv_ref[...] = jnp.where(hv >= 0.0, hv, a * hv)


@jax.jit
def kernel(uv_adj, vu_adj, ufea, vfea, Wu1, Wv1, Wv2, Wu2,
           u_fc_w, u_fc_b, v_fc_w, v_fc_b,
           u_fc2_w, u_fc2_b, v_fc2_w, v_fc2_b, prelu_a):
    ufc1a = u_fc_w[:, :H].T
    ufc1b = u_fc_w[:, H:].T
    vfc1a = v_fc_w[:, :H].T
    vfc1b = v_fc_w[:, H:].T
    ufc2 = u_fc2_w.T
    vfc2 = v_fc2_w.T
    a2d = jnp.reshape(prelu_a, (1, 1))

    adj_spec = pl.BlockSpec((BLK, V), lambda p, b: (b, 0))
    full = lambda shape: pl.BlockSpec(shape, lambda p, b: (0,) * len(shape))
    out_spec = pl.BlockSpec((BLK, H), lambda p, b: (b, 0))

    hu, hv = pl.pallas_call(
        _dgcn_kernel,
        grid=(2, NB),
        in_specs=[
            adj_spec,
            adj_spec,
            full((U, D)),
            full((V, D)),
            full((D, H)), full((D, H)),
            full((H, H)), full((H, H)),
            full((H, H)), full((D, H)), full((1, H)),
            full((H, H)), full((D, H)), full((1, H)),
            full((H, H)), full((1, H)),
            full((H, H)), full((1, H)),
            full((1, 1)),
        ],
        out_specs=[out_spec, out_spec],
        out_shape=[
            jax.ShapeDtypeStruct((U, H), jnp.float32),
            jax.ShapeDtypeStruct((V, H), jnp.float32),
        ],
        scratch_shapes=[
            pltpu.VMEM((U, H), jnp.float32),
            pltpu.VMEM((V, H), jnp.float32),
            pltpu.VMEM((V, H), jnp.float32),
            pltpu.VMEM((U, H), jnp.float32),
            pltpu.VMEM((V, H), jnp.float32),
            pltpu.VMEM((U, H), jnp.float32),
        ],
        compiler_params=pltpu.CompilerParams(
            dimension_semantics=("arbitrary", "arbitrary"),
        ),
    )(uv_adj, vu_adj, ufea, vfea, Wu1, Wv1, Wv2, Wu2,
      ufc1a, ufc1b, jnp.reshape(u_fc_b, (1, H)),
      vfc1a, vfc1b, jnp.reshape(v_fc_b, (1, H)),
      ufc2, jnp.reshape(u_fc2_b, (1, H)),
      vfc2, jnp.reshape(v_fc2_b, (1, H)),
      a2d)
    return (hu, hv)


# speedup vs baseline: 2.7410x; 2.7410x over previous
import functools

import jax
import jax.numpy as jnp
from jax.experimental import pallas as pl
from jax.experimental.pallas import tpu as pltpu

U = 4096
V = 4096
D = 128
H = 128
BLK = 256
NB = U // BLK

_PREC = jax.lax.Precision.DEFAULT


def _dot(a, b):
    return jax.lax.dot_general(
        a, b, (((1,), (0,)), ((), ())),
        precision=_PREC, preferred_element_type=jnp.float32)


def _dgcn_kernel(
    uv_adj_ref, vu_adj_ref, ufea_ref, vfea_ref,
    Wu1_ref, Wv1_ref, Wv2_ref, Wu2_ref,
    ufc1a_ref, ufc1b_ref, ufc1bias_ref,
    vfc1a_ref, vfc1b_ref, vfc1bias_ref,
    ufc2_ref, ufc2bias_ref, vfc2_ref, vfc2bias_ref,
    a_ref,
    hu_ref, hv_ref,
    su_s, sv_s, vu_s, uv_s, tv_s, tu_s,
):
    p = pl.program_id(0)
    b = pl.program_id(1)
    rows = pl.ds(b * BLK, BLK)

    @pl.when(jnp.logical_and(p == 0, b == 0))
    def _init_supports():
        su_s[...] = _dot(ufea_ref[...], Wu1_ref[...])
        sv_s[...] = _dot(vfea_ref[...], Wv1_ref[...])

    @pl.when(p == 0)
    def _phase0():
        vu_s[rows, :] = jnp.maximum(_dot(vu_adj_ref[...], su_s[...]), 0.0)
        uv_s[rows, :] = jnp.maximum(_dot(uv_adj_ref[...], sv_s[...]), 0.0)

    @pl.when(jnp.logical_and(p == 1, b == 0))
    def _init_t():
        tv_s[...] = _dot(vu_s[...], Wv2_ref[...])
        tu_s[...] = _dot(uv_s[...], Wu2_ref[...])

    @pl.when(p == 1)
    def _phase1():
        a = a_ref[0, 0]
        uv2 = jnp.maximum(_dot(uv_adj_ref[...], tv_s[...]), 0.0)
        vu2 = jnp.maximum(_dot(vu_adj_ref[...], tu_s[...]), 0.0)

        hu = _dot(uv2, ufc1a_ref[...]) + _dot(ufea_ref[rows, :], ufc1b_ref[...])
        hu = jnp.maximum(hu + ufc1bias_ref[...], 0.0)
        hu = _dot(hu, ufc2_ref[...]) + ufc2bias_ref[...]
        hu_ref[...] = jnp.where(hu >= 0.0, hu, a * hu)

        hv = _dot(vu2, vfc1a_ref[...]) + _dot(vfea_ref[rows, :], vfc1b_ref[...])
        hv = jnp.maximum(hv + vfc1bias_ref[...], 0.0)
        hv = _dot(hv, vfc2_ref[...]) + vfc2bias_ref[...]
        hv_ref[...] = jnp.where(hv >= 0.0, hv, a * hv)


@jax.jit
def kernel(uv_adj, vu_adj, ufea, vfea, Wu1, Wv1, Wv2, Wu2,
           u_fc_w, u_fc_b, v_fc_w, v_fc_b,
           u_fc2_w, u_fc2_b, v_fc2_w, v_fc2_b, prelu_a):
    ufc1a = u_fc_w[:, :H].T
    ufc1b = u_fc_w[:, H:].T
    vfc1a = v_fc_w[:, :H].T
    vfc1b = v_fc_w[:, H:].T
    ufc2 = u_fc2_w.T
    vfc2 = v_fc2_w.T
    a2d = jnp.reshape(prelu_a, (1, 1))

    adj_spec = pl.BlockSpec((BLK, V), lambda p, b: (b, 0))
    full = lambda shape: pl.BlockSpec(shape, lambda p, b: (0,) * len(shape))
    out_spec = pl.BlockSpec((BLK, H), lambda p, b: (b, 0))

    hu, hv = pl.pallas_call(
        _dgcn_kernel,
        grid=(2, NB),
        in_specs=[
            adj_spec,
            adj_spec,
            full((U, D)),
            full((V, D)),
            full((D, H)), full((D, H)),
            full((H, H)), full((H, H)),
            full((H, H)), full((D, H)), full((1, H)),
            full((H, H)), full((D, H)), full((1, H)),
            full((H, H)), full((1, H)),
            full((H, H)), full((1, H)),
            full((1, 1)),
        ],
        out_specs=[out_spec, out_spec],
        out_shape=[
            jax.ShapeDtypeStruct((U, H), jnp.float32),
            jax.ShapeDtypeStruct((V, H), jnp.float32),
        ],
        scratch_shapes=[
            pltpu.VMEM((U, H), jnp.float32),
            pltpu.VMEM((V, H), jnp.float32),
            pltpu.VMEM((V, H), jnp.float32),
            pltpu.VMEM((U, H), jnp.float32),
            pltpu.VMEM((V, H), jnp.float32),
            pltpu.VMEM((U, H), jnp.float32),
        ],
        compiler_params=pltpu.CompilerParams(
            dimension_semantics=("arbitrary", "arbitrary"),
        ),
    )(uv_adj, vu_adj, ufea, vfea, Wu1, Wv1, Wv2, Wu2,
      ufc1a, ufc1b, jnp.reshape(u_fc_b, (1, H)),
      vfc1a, vfc1b, jnp.reshape(v_fc_b, (1, H)),
      ufc2, jnp.reshape(u_fc2_b, (1, H)),
      vfc2, jnp.reshape(v_fc2_b, (1, H)),
      a2d)
    return (hu, hv)


# speedup vs baseline: 2.8232x; 1.0300x over previous
import functools

import jax
import jax.numpy as jnp
from jax.experimental import pallas as pl
from jax.experimental.pallas import tpu as pltpu

U = 4096
V = 4096
D = 128
H = 128
BLK = 256
NB = U // BLK

_PREC = jax.lax.Precision.DEFAULT


def _dot(a, b):
    return jax.lax.dot_general(
        a, b, (((1,), (0,)), ((), ())),
        precision=_PREC, preferred_element_type=jnp.float32)


def _dgcn_kernel(
    uv_adj_ref, vu_adj_ref, ufea_ref, vfea_ref,
    Wu1_ref, Wv1_ref, Wv2_ref, Wu2_ref,
    ufc1a_ref, ufc1b_ref, ufc1bias_ref,
    vfc1a_ref, vfc1b_ref, vfc1bias_ref,
    ufc2_ref, ufc2bias_ref, vfc2_ref, vfc2bias_ref,
    a_ref,
    hu_ref, hv_ref,
    su_s, sbv_s, vu_s, uv_s, tu_s,
):
    p = pl.program_id(0)
    b = pl.program_id(1)
    rows = pl.ds(b * BLK, BLK)

    @pl.when(jnp.logical_and(p == 0, b == 0))
    def _init_supports():
        su_s[...] = _dot(ufea_ref[...], Wu1_ref[...])
        sbv_s[:, :H] = _dot(vfea_ref[...], Wv1_ref[...])

    @pl.when(p == 0)
    def _phase0():
        vu_s[rows, :] = jnp.maximum(_dot(vu_adj_ref[...], su_s[...]), 0.0)

    @pl.when(jnp.logical_and(p == 1, b == 0))
    def _init_tv():
        sbv_s[:, H:] = _dot(vu_s[...], Wv2_ref[...])

    @pl.when(p == 1)
    def _phase1():
        a = a_ref[0, 0]
        st = jnp.maximum(_dot(uv_adj_ref[...], sbv_s[...]), 0.0)
        uv_s[rows, :] = st[:, :H]
        uv2 = st[:, H:]
        hu = _dot(uv2, ufc1a_ref[...]) + _dot(ufea_ref[rows, :], ufc1b_ref[...])
        hu = jnp.maximum(hu + ufc1bias_ref[...], 0.0)
        hu = _dot(hu, ufc2_ref[...]) + ufc2bias_ref[...]
        hu_ref[...] = jnp.where(hu >= 0.0, hu, a * hu)

    @pl.when(jnp.logical_and(p == 2, b == 0))
    def _init_tu():
        tu_s[...] = _dot(uv_s[...], Wu2_ref[...])

    @pl.when(p == 2)
    def _phase2():
        a = a_ref[0, 0]
        vu2 = jnp.maximum(_dot(vu_adj_ref[...], tu_s[...]), 0.0)
        hv = _dot(vu2, vfc1a_ref[...]) + _dot(vfea_ref[rows, :], vfc1b_ref[...])
        hv = jnp.maximum(hv + vfc1bias_ref[...], 0.0)
        hv = _dot(hv, vfc2_ref[...]) + vfc2bias_ref[...]
        hv_ref[...] = jnp.where(hv >= 0.0, hv, a * hv)


@jax.jit
def kernel(uv_adj, vu_adj, ufea, vfea, Wu1, Wv1, Wv2, Wu2,
           u_fc_w, u_fc_b, v_fc_w, v_fc_b,
           u_fc2_w, u_fc2_b, v_fc2_w, v_fc2_b, prelu_a):
    ufc1a = u_fc_w[:, :H].T
    ufc1b = u_fc_w[:, H:].T
    vfc1a = v_fc_w[:, :H].T
    vfc1b = v_fc_w[:, H:].T
    ufc2 = u_fc2_w.T
    vfc2 = v_fc2_w.T
    a2d = jnp.reshape(prelu_a, (1, 1))

    uv_adj_spec = pl.BlockSpec(
        (BLK, V), lambda p, b: (jnp.where(p == 0, 0, jnp.where(p == 1, b, NB - 1)), 0))
    vu_adj_spec = pl.BlockSpec(
        (BLK, U), lambda p, b: (jnp.where(p == 1, NB - 1, b), 0))
    full = lambda shape: pl.BlockSpec(shape, lambda p, b: (0,) * len(shape))
    hu_spec = pl.BlockSpec(
        (BLK, H), lambda p, b: (jnp.where(p == 0, 0, jnp.where(p == 1, b, NB - 1)), 0))
    hv_spec = pl.BlockSpec(
        (BLK, H), lambda p, b: (jnp.where(p == 2, b, 0), 0))

    hu, hv = pl.pallas_call(
        _dgcn_kernel,
        grid=(3, NB),
        in_specs=[
            uv_adj_spec,
            vu_adj_spec,
            full((U, D)),
            full((V, D)),
            full((D, H)), full((D, H)),
            full((H, H)), full((H, H)),
            full((H, H)), full((D, H)), full((1, H)),
            full((H, H)), full((D, H)), full((1, H)),
            full((H, H)), full((1, H)),
            full((H, H)), full((1, H)),
            full((1, 1)),
        ],
        out_specs=[hu_spec, hv_spec],
        out_shape=[
            jax.ShapeDtypeStruct((U, H), jnp.float32),
            jax.ShapeDtypeStruct((V, H), jnp.float32),
        ],
        scratch_shapes=[
            pltpu.VMEM((U, H), jnp.float32),
            pltpu.VMEM((V, 2 * H), jnp.float32),
            pltpu.VMEM((V, H), jnp.float32),
            pltpu.VMEM((U, H), jnp.float32),
            pltpu.VMEM((U, H), jnp.float32),
        ],
        compiler_params=pltpu.CompilerParams(
            dimension_semantics=("arbitrary", "arbitrary"),
        ),
    )(uv_adj, vu_adj, ufea, vfea, Wu1, Wv1, Wv2, Wu2,
      ufc1a, ufc1b, jnp.reshape(u_fc_b, (1, H)),
      vfc1a, vfc1b, jnp.reshape(v_fc_b, (1, H)),
      ufc2, jnp.reshape(u_fc2_b, (1, H)),
      vfc2, jnp.reshape(v_fc2_b, (1, H)),
      a2d)
    return (hu, hv)


# speedup vs baseline: 3.3574x; 1.1892x over previous
import functools

import jax
import jax.numpy as jnp
from jax.experimental import pallas as pl
from jax.experimental.pallas import tpu as pltpu

U = 4096
V = 4096
D = 128
H = 128
BLK = 512
NB = U // BLK

_PREC = jax.lax.Precision.DEFAULT


def _dot(a, b):
    return jax.lax.dot_general(
        a, b, (((1,), (0,)), ((), ())),
        precision=_PREC, preferred_element_type=jnp.float32)


def _dgcn_kernel(
    uv_adj_ref, vu_adj_ref, ufea_ref, vfea_ref,
    Wu1_ref, Wv1_ref, Wv2_ref, Wu2_ref,
    ufc1a_ref, ufc1b_ref, ufc1bias_ref,
    vfc1a_ref, vfc1b_ref, vfc1bias_ref,
    ufc2_ref, ufc2bias_ref, vfc2_ref, vfc2bias_ref,
    a_ref,
    hu_ref, hv_ref,
    su_s, sbv_s, vu_s, uv_s, tu_s,
):
    p = pl.program_id(0)
    b = pl.program_id(1)
    rows = pl.ds(b * BLK, BLK)

    @pl.when(jnp.logical_and(p == 0, b == 0))
    def _init_supports():
        su_s[...] = _dot(ufea_ref[...], Wu1_ref[...])
        sbv_s[:, :H] = _dot(vfea_ref[...], Wv1_ref[...])

    @pl.when(p == 0)
    def _phase0():
        vu_s[rows, :] = jnp.maximum(_dot(vu_adj_ref[...], su_s[...]), 0.0)

    @pl.when(jnp.logical_and(p == 1, b == 0))
    def _init_tv():
        sbv_s[:, H:] = _dot(vu_s[...], Wv2_ref[...])

    @pl.when(p == 1)
    def _phase1():
        a = a_ref[0, 0]
        st = jnp.maximum(_dot(uv_adj_ref[...], sbv_s[...]), 0.0)
        uv_s[rows, :] = st[:, :H]
        uv2 = st[:, H:]
        hu = _dot(uv2, ufc1a_ref[...]) + _dot(ufea_ref[rows, :], ufc1b_ref[...])
        hu = jnp.maximum(hu + ufc1bias_ref[...], 0.0)
        hu = _dot(hu, ufc2_ref[...]) + ufc2bias_ref[...]
        hu_ref[...] = jnp.where(hu >= 0.0, hu, a * hu)

    @pl.when(jnp.logical_and(p == 2, b == 0))
    def _init_tu():
        tu_s[...] = _dot(uv_s[...], Wu2_ref[...])

    @pl.when(p == 2)
    def _phase2():
        a = a_ref[0, 0]
        vu2 = jnp.maximum(_dot(vu_adj_ref[...], tu_s[...]), 0.0)
        hv = _dot(vu2, vfc1a_ref[...]) + _dot(vfea_ref[rows, :], vfc1b_ref[...])
        hv = jnp.maximum(hv + vfc1bias_ref[...], 0.0)
        hv = _dot(hv, vfc2_ref[...]) + vfc2bias_ref[...]
        hv_ref[...] = jnp.where(hv >= 0.0, hv, a * hv)


@jax.jit
def kernel(uv_adj, vu_adj, ufea, vfea, Wu1, Wv1, Wv2, Wu2,
           u_fc_w, u_fc_b, v_fc_w, v_fc_b,
           u_fc2_w, u_fc2_b, v_fc2_w, v_fc2_b, prelu_a):
    ufc1a = u_fc_w[:, :H].T
    ufc1b = u_fc_w[:, H:].T
    vfc1a = v_fc_w[:, :H].T
    vfc1b = v_fc_w[:, H:].T
    ufc2 = u_fc2_w.T
    vfc2 = v_fc2_w.T
    a2d = jnp.reshape(prelu_a, (1, 1))

    uv_adj_spec = pl.BlockSpec(
        (BLK, V), lambda p, b: (jnp.where(p == 0, 0, jnp.where(p == 1, b, NB - 1)), 0))
    vu_adj_spec = pl.BlockSpec(
        (BLK, U), lambda p, b: (jnp.where(p == 1, NB - 1, b), 0))
    full = lambda shape: pl.BlockSpec(shape, lambda p, b: (0,) * len(shape))
    hu_spec = pl.BlockSpec(
        (BLK, H), lambda p, b: (jnp.where(p == 0, 0, jnp.where(p == 1, b, NB - 1)), 0))
    hv_spec = pl.BlockSpec(
        (BLK, H), lambda p, b: (jnp.where(p == 2, b, 0), 0))

    hu, hv = pl.pallas_call(
        _dgcn_kernel,
        grid=(3, NB),
        in_specs=[
            uv_adj_spec,
            vu_adj_spec,
            full((U, D)),
            full((V, D)),
            full((D, H)), full((D, H)),
            full((H, H)), full((H, H)),
            full((H, H)), full((D, H)), full((1, H)),
            full((H, H)), full((D, H)), full((1, H)),
            full((H, H)), full((1, H)),
            full((H, H)), full((1, H)),
            full((1, 1)),
        ],
        out_specs=[hu_spec, hv_spec],
        out_shape=[
            jax.ShapeDtypeStruct((U, H), jnp.float32),
            jax.ShapeDtypeStruct((V, H), jnp.float32),
        ],
        scratch_shapes=[
            pltpu.VMEM((U, H), jnp.float32),
            pltpu.VMEM((V, 2 * H), jnp.float32),
            pltpu.VMEM((V, H), jnp.float32),
            pltpu.VMEM((U, H), jnp.float32),
            pltpu.VMEM((U, H), jnp.float32),
        ],
        compiler_params=pltpu.CompilerParams(
            dimension_semantics=("arbitrary", "arbitrary"),
        ),
    )(uv_adj, vu_adj, ufea, vfea, Wu1, Wv1, Wv2, Wu2,
      ufc1a, ufc1b, jnp.reshape(u_fc_b, (1, H)),
      vfc1a, vfc1b, jnp.reshape(v_fc_b, (1, H)),
      ufc2, jnp.reshape(u_fc2_b, (1, H)),
      vfc2, jnp.reshape(v_fc2_b, (1, H)),
      a2d)
    return (hu, hv)


# speedup vs baseline: 3.3622x; 1.0014x over previous
import functools

import jax
import jax.numpy as jnp
from jax.experimental import pallas as pl
from jax.experimental.pallas import tpu as pltpu

U = 4096
V = 4096
D = 128
H = 128
BLK = 512
NB = U // BLK

_PREC = jax.lax.Precision.DEFAULT


def _dot(a, b):
    return jax.lax.dot_general(
        a, b, (((1,), (0,)), ((), ())),
        precision=_PREC, preferred_element_type=jnp.float32)


def _dgcn_kernel(
    uv_adj_ref, vu_adj_ref, ufea_ref, vfea_ref,
    Wu1_ref, Wv1_ref, Wv2_ref, Wu2_ref,
    ufc1a_ref, ufc1b_ref, ufc1bias_ref,
    vfc1a_ref, vfc1b_ref, vfc1bias_ref,
    ufc2_ref, ufc2bias_ref, vfc2_ref, vfc2bias_ref,
    a_ref,
    hu_ref, hv_ref,
    su_s, sbv_s, vu_s, uv_s, tu_s,
):
    p = pl.program_id(0)
    b = pl.program_id(1)
    rows = pl.ds(b * BLK, BLK)

    @pl.when(jnp.logical_and(p == 0, b == 0))
    def _init_supports():
        su_s[...] = _dot(ufea_ref[...], Wu1_ref[...]).astype(jnp.bfloat16)
        sbv_s[:, :H] = _dot(vfea_ref[...], Wv1_ref[...]).astype(jnp.bfloat16)

    @pl.when(p == 0)
    def _phase0():
        adj = vu_adj_ref[...].astype(jnp.bfloat16)
        vu_s[rows, :] = jnp.maximum(_dot(adj, su_s[...]), 0.0).astype(jnp.bfloat16)

    @pl.when(jnp.logical_and(p == 1, b == 0))
    def _init_tv():
        sbv_s[:, H:] = _dot(vu_s[...], Wv2_ref[...].astype(jnp.bfloat16)
                            ).astype(jnp.bfloat16)

    @pl.when(p == 1)
    def _phase1():
        a = a_ref[0, 0]
        adj = uv_adj_ref[...].astype(jnp.bfloat16)
        st = jnp.maximum(_dot(adj, sbv_s[...]), 0.0)
        uv_s[rows, :] = st[:, :H].astype(jnp.bfloat16)
        uv2 = st[:, H:]
        hu = _dot(uv2, ufc1a_ref[...]) + _dot(ufea_ref[rows, :], ufc1b_ref[...])
        hu = jnp.maximum(hu + ufc1bias_ref[...], 0.0)
        hu = _dot(hu, ufc2_ref[...]) + ufc2bias_ref[...]
        hu_ref[...] = jnp.where(hu >= 0.0, hu, a * hu)

    @pl.when(jnp.logical_and(p == 2, b == 0))
    def _init_tu():
        tu_s[...] = _dot(uv_s[...], Wu2_ref[...].astype(jnp.bfloat16)
                         ).astype(jnp.bfloat16)

    @pl.when(p == 2)
    def _phase2():
        a = a_ref[0, 0]
        adj = vu_adj_ref[...].astype(jnp.bfloat16)
        vu2 = jnp.maximum(_dot(adj, tu_s[...]), 0.0)
        hv = _dot(vu2, vfc1a_ref[...]) + _dot(vfea_ref[rows, :], vfc1b_ref[...])
        hv = jnp.maximum(hv + vfc1bias_ref[...], 0.0)
        hv = _dot(hv, vfc2_ref[...]) + vfc2bias_ref[...]
        hv_ref[...] = jnp.where(hv >= 0.0, hv, a * hv)


@jax.jit
def kernel(uv_adj, vu_adj, ufea, vfea, Wu1, Wv1, Wv2, Wu2,
           u_fc_w, u_fc_b, v_fc_w, v_fc_b,
           u_fc2_w, u_fc2_b, v_fc2_w, v_fc2_b, prelu_a):
    ufc1a = u_fc_w[:, :H].T
    ufc1b = u_fc_w[:, H:].T
    vfc1a = v_fc_w[:, :H].T
    vfc1b = v_fc_w[:, H:].T
    ufc2 = u_fc2_w.T
    vfc2 = v_fc2_w.T
    a2d = jnp.reshape(prelu_a, (1, 1))

    uv_adj_spec = pl.BlockSpec(
        (BLK, V), lambda p, b: (jnp.where(p == 0, 0, jnp.where(p == 1, b, NB - 1)), 0))
    vu_adj_spec = pl.BlockSpec(
        (BLK, U), lambda p, b: (jnp.where(p == 1, NB - 1, b), 0))
    full = lambda shape: pl.BlockSpec(shape, lambda p, b: (0,) * len(shape))
    hu_spec = pl.BlockSpec(
        (BLK, H), lambda p, b: (jnp.where(p == 0, 0, jnp.where(p == 1, b, NB - 1)), 0))
    hv_spec = pl.BlockSpec(
        (BLK, H), lambda p, b: (jnp.where(p == 2, b, 0), 0))

    hu, hv = pl.pallas_call(
        _dgcn_kernel,
        grid=(3, NB),
        in_specs=[
            uv_adj_spec,
            vu_adj_spec,
            full((U, D)),
            full((V, D)),
            full((D, H)), full((D, H)),
            full((H, H)), full((H, H)),
            full((H, H)), full((D, H)), full((1, H)),
            full((H, H)), full((D, H)), full((1, H)),
            full((H, H)), full((1, H)),
            full((H, H)), full((1, H)),
            full((1, 1)),
        ],
        out_specs=[hu_spec, hv_spec],
        out_shape=[
            jax.ShapeDtypeStruct((U, H), jnp.float32),
            jax.ShapeDtypeStruct((V, H), jnp.float32),
        ],
        scratch_shapes=[
            pltpu.VMEM((U, H), jnp.bfloat16),
            pltpu.VMEM((V, 2 * H), jnp.bfloat16),
            pltpu.VMEM((V, H), jnp.bfloat16),
            pltpu.VMEM((U, H), jnp.bfloat16),
            pltpu.VMEM((U, H), jnp.bfloat16),
        ],
        compiler_params=pltpu.CompilerParams(
            dimension_semantics=("arbitrary", "arbitrary"),
        ),
    )(uv_adj, vu_adj, ufea, vfea, Wu1, Wv1, Wv2, Wu2,
      ufc1a, ufc1b, jnp.reshape(u_fc_b, (1, H)),
      vfc1a, vfc1b, jnp.reshape(v_fc_b, (1, H)),
      ufc2, jnp.reshape(u_fc2_b, (1, H)),
      vfc2, jnp.reshape(v_fc2_b, (1, H)),
      a2d)
    return (hu, hv)


# speedup vs baseline: 3.7258x; 1.1082x over previous
import functools

import jax
import jax.numpy as jnp
from jax.experimental import pallas as pl
from jax.experimental.pallas import tpu as pltpu

U = 4096
V = 4096
D = 128
H = 128
BLK = 512
NB = U // BLK

_PREC = jax.lax.Precision.DEFAULT


def _dot(a, b):
    return jax.lax.dot_general(
        a, b, (((1,), (0,)), ((), ())),
        precision=_PREC, preferred_element_type=jnp.float32)


def _dotT(a, b):
    return jax.lax.dot_general(
        a, b, (((1,), (1,)), ((), ())),
        precision=_PREC, preferred_element_type=jnp.float32)


def _dgcn_kernel(
    uv_adj_ref, vu_adj_ref, ufea_ref, vfea_ref,
    Wu1_ref, Wv1_ref, Wv2_ref, Wu2_ref,
    ufc1_ref, ufc1bias_ref, vfc1_ref, vfc1bias_ref,
    ufc2_ref, ufc2bias_ref, vfc2_ref, vfc2bias_ref,
    a_ref,
    hu_ref, hv_ref,
    su_s, sbv_s, vu_s, uv_s, tu_s,
):
    p = pl.program_id(0)
    b = pl.program_id(1)
    rows = pl.ds(b * BLK, BLK)

    @pl.when(jnp.logical_and(p == 0, b == 0))
    def _init_supports():
        su_s[...] = _dot(ufea_ref[...], Wu1_ref[...]).astype(jnp.bfloat16)
        sbv_s[:, :H] = _dot(vfea_ref[...], Wv1_ref[...]).astype(jnp.bfloat16)

    @pl.when(p == 0)
    def _phase0():
        adj = vu_adj_ref[...].astype(jnp.bfloat16)
        vu_s[rows, :] = jnp.maximum(_dot(adj, su_s[...]), 0.0).astype(jnp.bfloat16)

    @pl.when(jnp.logical_and(p == 1, b == 0))
    def _init_tv():
        sbv_s[:, H:] = _dot(vu_s[...], Wv2_ref[...].astype(jnp.bfloat16)
                            ).astype(jnp.bfloat16)

    @pl.when(p == 1)
    def _phase1():
        a = a_ref[0, 0]
        adj = uv_adj_ref[...].astype(jnp.bfloat16)
        st = jnp.maximum(_dot(adj, sbv_s[...]), 0.0)
        uv_s[rows, :] = st[:, :H].astype(jnp.bfloat16)
        uv2 = st[:, H:]
        hu = (_dotT(uv2, ufc1_ref[:, :H])
              + _dotT(ufea_ref[rows, :], ufc1_ref[:, H:]))
        hu = jnp.maximum(hu + ufc1bias_ref[...], 0.0)
        hu = _dotT(hu, ufc2_ref[...]) + ufc2bias_ref[...]
        hu_ref[...] = jnp.where(hu >= 0.0, hu, a * hu)

    @pl.when(jnp.logical_and(p == 2, b == 0))
    def _init_tu():
        tu_s[...] = _dot(uv_s[...], Wu2_ref[...].astype(jnp.bfloat16)
                         ).astype(jnp.bfloat16)

    @pl.when(p == 2)
    def _phase2():
        a = a_ref[0, 0]
        adj = vu_adj_ref[...].astype(jnp.bfloat16)
        vu2 = jnp.maximum(_dot(adj, tu_s[...]), 0.0)
        hv = (_dotT(vu2, vfc1_ref[:, :H])
              + _dotT(vfea_ref[rows, :], vfc1_ref[:, H:]))
        hv = jnp.maximum(hv + vfc1bias_ref[...], 0.0)
        hv = _dotT(hv, vfc2_ref[...]) + vfc2bias_ref[...]
        hv_ref[...] = jnp.where(hv >= 0.0, hv, a * hv)


@jax.jit
def kernel(uv_adj, vu_adj, ufea, vfea, Wu1, Wv1, Wv2, Wu2,
           u_fc_w, u_fc_b, v_fc_w, v_fc_b,
           u_fc2_w, u_fc2_b, v_fc2_w, v_fc2_b, prelu_a):
    a2d = jnp.reshape(prelu_a, (1, 1))

    uv_adj_spec = pl.BlockSpec(
        (BLK, V), lambda p, b: (jnp.where(p == 0, 0, jnp.where(p == 1, b, NB - 1)), 0))
    vu_adj_spec = pl.BlockSpec(
        (BLK, U), lambda p, b: (jnp.where(p == 1, NB - 1, b), 0))
    full = lambda shape: pl.BlockSpec(shape, lambda p, b: (0,) * len(shape))
    hu_spec = pl.BlockSpec(
        (BLK, H), lambda p, b: (jnp.where(p == 0, 0, jnp.where(p == 1, b, NB - 1)), 0))
    hv_spec = pl.BlockSpec(
        (BLK, H), lambda p, b: (jnp.where(p == 2, b, 0), 0))

    hu, hv = pl.pallas_call(
        _dgcn_kernel,
        grid=(3, NB),
        in_specs=[
            uv_adj_spec,
            vu_adj_spec,
            full((U, D)),
            full((V, D)),
            full((D, H)), full((D, H)),
            full((H, H)), full((H, H)),
            full((H, H + D)), full((1, H)),
            full((H, H + D)), full((1, H)),
            full((H, H)), full((1, H)),
            full((H, H)), full((1, H)),
            full((1, 1)),
        ],
        out_specs=[hu_spec, hv_spec],
        out_shape=[
            jax.ShapeDtypeStruct((U, H), jnp.float32),
            jax.ShapeDtypeStruct((V, H), jnp.float32),
        ],
        scratch_shapes=[
            pltpu.VMEM((U, H), jnp.bfloat16),
            pltpu.VMEM((V, 2 * H), jnp.bfloat16),
            pltpu.VMEM((V, H), jnp.bfloat16),
            pltpu.VMEM((U, H), jnp.bfloat16),
            pltpu.VMEM((U, H), jnp.bfloat16),
        ],
        compiler_params=pltpu.CompilerParams(
            dimension_semantics=("arbitrary", "arbitrary"),
        ),
    )(uv_adj, vu_adj, ufea, vfea, Wu1, Wv1, Wv2, Wu2,
      u_fc_w, jnp.reshape(u_fc_b, (1, H)),
      v_fc_w, jnp.reshape(v_fc_b, (1, H)),
      u_fc2_w, jnp.reshape(u_fc2_b, (1, H)),
      v_fc2_w, jnp.reshape(v_fc2_b, (1, H)),
      a2d)
    return (hu, hv)
